# Initial kernel scaffold; baseline (speedup 1.0000x reference)
import jax
import jax.numpy as jnp
from jax.experimental import pallas as pl


def kernel(x, r_ij, neighbors, neighbor_mask, f_ij, W1, b1, W2, b2, W_in, W_out, b_out, W_d, b_d):
    raise NotImplementedError("write your pallas kernel here")


# speedup vs baseline: 11.6666x; 11.6666x over previous
import functools

import jax
import jax.numpy as jnp
from jax import lax
from jax.experimental import pallas as pl
from jax.experimental.pallas import tpu as pltpu
from jax.experimental.pallas import tpu_sc as plsc

F32 = jnp.float32
CUTOFF_R = 5.0
LN2 = 0.6931471805599453


def _ssp(v):
    return jnp.logaddexp(v, 0.0) - LN2


def _in2f_kernel(x_ref, w_ref, o_ref):
    o_ref[...] = jnp.dot(x_ref[...], w_ref[...], preferred_element_type=F32)


def _in2f(x2d, w_in):
    m, k = x2d.shape
    f = w_in.shape[1]
    g = 8
    return pl.pallas_call(
        _in2f_kernel,
        grid=(g,),
        in_specs=[
            pl.BlockSpec((m // g, k), lambda i: (i, 0)),
            pl.BlockSpec((k, f), lambda i: (0, 0)),
        ],
        out_specs=pl.BlockSpec((m // g, f), lambda i: (i, 0)),
        out_shape=jax.ShapeDtypeStruct((m, f), F32),
    )(x2d, w_in)


def _sc_gather(y2d, idx3):
    info = plsc.get_sparse_core_info()
    nc, ns = info.num_cores, info.num_subcores
    nw = nc * ns
    nch, ch = idx3.shape[1], idx3.shape[2]
    fdim = y2d.shape[1]
    assert idx3.shape[0] == nw
    mesh = plsc.VectorSubcoreMesh(core_axis_name="c", subcore_axis_name="s")
    e = nw * nch * ch

    @functools.partial(
        pl.kernel,
        mesh=mesh,
        out_type=jax.ShapeDtypeStruct((e, fdim), F32),
        scratch_types=[
            pltpu.VMEM((ch,), jnp.int32),
            pltpu.VMEM((ch, fdim), F32),
            pltpu.SemaphoreType.DMA,
        ],
    )
    def gk(y_hbm, idx_hbm, out_hbm, idx_v, rows_v, sem):
        wid = lax.axis_index("s") * nc + lax.axis_index("c")

        def body(j, carry):
            pltpu.sync_copy(idx_hbm.at[wid, j], idx_v)
            pltpu.async_copy(y_hbm.at[idx_v], rows_v, sem).wait()
            pltpu.sync_copy(rows_v, out_hbm.at[pl.ds((wid * nch + j) * ch, ch)])
            return carry

        lax.fori_loop(0, nch, body, 0)

    return gk(y2d, idx3)


def _fused_kernel(tn, nbh, f_ref, r_ref, m_ref, yg_ref, w1_ref, b1_ref,
                  w2_ref, b2_ref, wo_ref, bo_ref, wd_ref, bd_ref, o_ref):
    ff = yg_ref.shape[1]
    h = jnp.dot(f_ref[...], w1_ref[...], preferred_element_type=F32) + b1_ref[...]
    h = _ssp(h)
    w = jnp.dot(h, w2_ref[...], preferred_element_type=F32) + b2_ref[...]
    c = (r_ref[...] <= CUTOFF_R).astype(F32) * m_ref[...]
    w3 = w.reshape(tn, nbh, ff) * c[:, :, None]
    agg = jnp.sum(w3 * yg_ref[...].reshape(tn, nbh, ff), axis=1)
    v = _ssp(jnp.dot(agg, wo_ref[...], preferred_element_type=F32) + bo_ref[...])
    o_ref[...] = jnp.dot(v, wd_ref[...], preferred_element_type=F32) + bd_ref[...]


def _fused(f2, r2, m2, y_g, w1, b1, w2, b2, w_out, b_out, w_d, b_d):
    bn, nbh = r2.shape
    s = f2.shape[1]
    ff = y_g.shape[1]
    tn = 128
    g = bn // tn
    body = functools.partial(_fused_kernel, tn, nbh)
    return pl.pallas_call(
        body,
        grid=(g,),
        in_specs=[
            pl.BlockSpec((tn * nbh, s), lambda i: (i, 0)),
            pl.BlockSpec((tn, nbh), lambda i: (i, 0)),
            pl.BlockSpec((tn, nbh), lambda i: (i, 0)),
            pl.BlockSpec((tn * nbh, ff), lambda i: (i, 0)),
            pl.BlockSpec((s, ff), lambda i: (0, 0)),
            pl.BlockSpec((1, ff), lambda i: (0, 0)),
            pl.BlockSpec((ff, ff), lambda i: (0, 0)),
            pl.BlockSpec((1, ff), lambda i: (0, 0)),
            pl.BlockSpec((ff, ff), lambda i: (0, 0)),
            pl.BlockSpec((1, ff), lambda i: (0, 0)),
            pl.BlockSpec((ff, ff), lambda i: (0, 0)),
            pl.BlockSpec((1, ff), lambda i: (0, 0)),
        ],
        out_specs=pl.BlockSpec((tn, ff), lambda i: (i, 0)),
        out_shape=jax.ShapeDtypeStruct((bn, ff), F32),
    )(f2, r2, m2, y_g, w1, b1, w2, b2, w_out, b_out, w_d, b_d)


def kernel(x, r_ij, neighbors, neighbor_mask, f_ij, W1, b1, W2, b2,
           W_in, W_out, b_out, W_d, b_d):
    b, n, f = x.shape
    nbh = neighbors.shape[2]
    s = f_ij.shape[3]
    e = b * n * nbh

    y2d = _in2f(x.reshape(b * n, f), W_in)

    base = (jnp.arange(b, dtype=jnp.int32) * n)[:, None, None]
    flat_idx = (neighbors.astype(jnp.int32) + base).reshape(-1)
    info = plsc.get_sparse_core_info()
    nw = info.num_cores * info.num_subcores
    ch = 128
    idx3 = flat_idx.reshape(nw, e // (nw * ch), ch)
    y_g = _sc_gather(y2d, idx3)

    out = _fused(
        f_ij.reshape(e, s),
        r_ij.reshape(b * n, nbh),
        neighbor_mask.reshape(b * n, nbh),
        y_g,
        W1, b1.reshape(1, -1), W2, b2.reshape(1, -1),
        W_out, b_out.reshape(1, -1), W_d, b_d.reshape(1, -1),
    )
    return out.reshape(b, n, f)


# speedup vs baseline: 14.8609x; 1.2738x over previous
import functools

import jax
import jax.numpy as jnp
from jax import lax
from jax.experimental import pallas as pl
from jax.experimental.pallas import tpu as pltpu
from jax.experimental.pallas import tpu_sc as plsc

F32 = jnp.float32
CUTOFF_R = 5.0
LN2 = 0.6931471805599453


def _ssp(v):
    return jnp.logaddexp(v, 0.0) - LN2


def _in2f_kernel(x_ref, w_ref, o_ref):
    o_ref[...] = jnp.dot(x_ref[...], w_ref[...], preferred_element_type=F32)


def _in2f(x2d, w_in):
    m, k = x2d.shape
    f = w_in.shape[1]
    g = 8
    return pl.pallas_call(
        _in2f_kernel,
        grid=(g,),
        in_specs=[
            pl.BlockSpec((m // g, k), lambda i: (i, 0)),
            pl.BlockSpec((k, f), lambda i: (0, 0)),
        ],
        out_specs=pl.BlockSpec((m // g, f), lambda i: (i, 0)),
        out_shape=jax.ShapeDtypeStruct((m, f), F32),
    )(x2d, w_in)


def _sc_gather(y2d, idx3):
    info = plsc.get_sparse_core_info()
    nc, ns = info.num_cores, info.num_subcores
    nw = nc * ns
    nch, ch = idx3.shape[1], idx3.shape[2]
    fdim = y2d.shape[1]
    assert idx3.shape[0] == nw
    nbuf = 4
    assert nch % nbuf == 0
    mesh = plsc.VectorSubcoreMesh(core_axis_name="c", subcore_axis_name="s")
    e = nw * nch * ch

    @functools.partial(
        pl.kernel,
        mesh=mesh,
        out_type=jax.ShapeDtypeStruct((e, fdim), F32),
        scratch_types=[pltpu.VMEM((nch, ch), jnp.int32)]
        + [pltpu.VMEM((ch, fdim), F32) for _ in range(nbuf)]
        + [pltpu.SemaphoreType.DMA for _ in range(2 * nbuf)],
    )
    def gk(y_hbm, idx_hbm, out_hbm, idx_v, *bufs_and_sems):
        rows = bufs_and_sems[:nbuf]
        gsem = bufs_and_sems[nbuf:2 * nbuf]
        wsem = bufs_and_sems[2 * nbuf:]
        wid = lax.axis_index("s") * nc + lax.axis_index("c")
        pltpu.sync_copy(idx_hbm.at[wid], idx_v)
        for b in range(nbuf):
            pltpu.async_copy(y_hbm.at[idx_v.at[b]], rows[b], gsem[b])

        def body(g, carry):
            for b in range(nbuf):
                j = g * nbuf + b
                dst = out_hbm.at[pl.ds((wid * nch + j) * ch, ch)]
                pltpu.make_async_copy(y_hbm.at[idx_v.at[j]], rows[b],
                                      gsem[b]).wait()
                pltpu.async_copy(rows[b], dst, wsem[b])
                nxt = j + nbuf

                @pl.when(nxt < nch)
                def _():
                    pltpu.make_async_copy(rows[b], dst, wsem[b]).wait()
                    pltpu.async_copy(y_hbm.at[idx_v.at[nxt]], rows[b],
                                     gsem[b])

                @pl.when(nxt >= nch)
                def _():
                    pltpu.make_async_copy(rows[b], dst, wsem[b]).wait()

            return carry

        lax.fori_loop(0, nch // nbuf, body, 0)

    return gk(y2d, idx3)


def _fused_kernel(tn, nbh, f_ref, r_ref, m_ref, yg_ref, w1_ref, b1_ref,
                  w2_ref, b2_ref, wo_ref, bo_ref, wd_ref, bd_ref, o_ref):
    ff = yg_ref.shape[1]
    h = jnp.dot(f_ref[...], w1_ref[...], preferred_element_type=F32) + b1_ref[...]
    h = _ssp(h)
    w = jnp.dot(h, w2_ref[...], preferred_element_type=F32) + b2_ref[...]
    c = (r_ref[...] <= CUTOFF_R).astype(F32) * m_ref[...]
    w3 = w.reshape(tn, nbh, ff) * c[:, :, None]
    yg = yg_ref[...].reshape(tn, nbh, ff)
    agg = jnp.sum(w3 * yg, axis=1)
    v = _ssp(jnp.dot(agg, wo_ref[...], preferred_element_type=F32) + bo_ref[...])
    o_ref[...] = jnp.dot(v, wd_ref[...], preferred_element_type=F32) + bd_ref[...]


def _fused(f2, r2, m2, y_g, w1, b1, w2, b2, w_out, b_out, w_d, b_d):
    bn, nbh = r2.shape
    s = f2.shape[1]
    fp = y_g.shape[1]
    ff = w_d.shape[0]
    tn = 128
    g = bn // tn
    body = functools.partial(_fused_kernel, tn, nbh)
    return pl.pallas_call(
        body,
        grid=(g,),
        in_specs=[
            pl.BlockSpec((tn * nbh, s), lambda i: (i, 0)),
            pl.BlockSpec((tn, nbh), lambda i: (i, 0)),
            pl.BlockSpec((tn, nbh), lambda i: (i, 0)),
            pl.BlockSpec((tn * nbh, fp), lambda i: (i, 0)),
            pl.BlockSpec((s, ff), lambda i: (0, 0)),
            pl.BlockSpec((1, ff), lambda i: (0, 0)),
            pl.BlockSpec((ff, ff), lambda i: (0, 0)),
            pl.BlockSpec((1, ff), lambda i: (0, 0)),
            pl.BlockSpec((ff, ff), lambda i: (0, 0)),
            pl.BlockSpec((1, ff), lambda i: (0, 0)),
            pl.BlockSpec((ff, ff), lambda i: (0, 0)),
            pl.BlockSpec((1, ff), lambda i: (0, 0)),
        ],
        out_specs=pl.BlockSpec((tn, ff), lambda i: (i, 0)),
        out_shape=jax.ShapeDtypeStruct((bn, ff), F32),
    )(f2, r2, m2, y_g, w1, b1, w2, b2, w_out, b_out, w_d, b_d)


def kernel(x, r_ij, neighbors, neighbor_mask, f_ij, W1, b1, W2, b2,
           W_in, W_out, b_out, W_d, b_d):
    b, n, f = x.shape
    nbh = neighbors.shape[2]
    s = f_ij.shape[3]
    e = b * n * nbh

    y2d = _in2f(x.reshape(b * n, f), W_in)

    base = (jnp.arange(b, dtype=jnp.int32) * n)[:, None, None]
    flat_idx = (neighbors.astype(jnp.int32) + base).reshape(-1)
    info = plsc.get_sparse_core_info()
    nw = info.num_cores * info.num_subcores
    ch = 128
    idx3 = flat_idx.reshape(nw, e // (nw * ch), ch)
    y_g = _sc_gather(y2d, idx3)

    out = _fused(
        f_ij.reshape(e, s),
        r_ij.reshape(b * n, nbh),
        neighbor_mask.reshape(b * n, nbh),
        y_g,
        W1, b1.reshape(1, -1), W2, b2.reshape(1, -1),
        W_out, b_out.reshape(1, -1), W_d, b_d.reshape(1, -1),
    )
    return out.reshape(b, n, f)
